# baseline (speedup 1.0000x reference)
import functools

import jax
import jax.numpy as jnp
from jax import lax
from jax.experimental import pallas as pl
from jax.experimental.pallas import tpu as pltpu
from jax.experimental.pallas import tpu_sc as plsc

VOCAB_N = 1000000
EMB_DIM = 64
ROW_W = 128
IDX_W = 128
NSLOT = 4

N_VBLK = VOCAB_N // IDX_W
N_WORKERS = 32
VBLK_MAIN = 7808
VBLK_PER_W = VBLK_MAIN // N_WORKERS


@jax.jit
def _relayout(tableT, tail):
    mesh = plsc.VectorSubcoreMesh(core_axis_name="c", subcore_axis_name="s")
    info = plsc.get_sparse_core_info()
    nc = info.num_cores

    def body(tabT_hbm, tail_hbm, out_hbm, src_v, dstT_v, sem_i, sem_o):
        wid = lax.axis_index("s") * nc + lax.axis_index("c")

        def blk_of(j):
            return j * N_WORKERS + wid

        def fire_in(blk, slot):
            pltpu.async_copy(tabT_hbm.at[:, pl.ds(blk * IDX_W, IDX_W)],
                             src_v.at[slot], sem_i.at[slot])

        def wait_in(slot):
            pltpu.make_async_copy(tabT_hbm.at[:, pl.ds(0, IDX_W)],
                                  src_v.at[slot], sem_i.at[slot]).wait()

        def fire_out(blk, slot):
            pltpu.async_copy(dstT_v.at[slot, :, pl.ds(0, ROW_W)],
                             out_hbm.at[pl.ds(blk * IDX_W, IDX_W)],
                             sem_o.at[slot])

        def wait_out(slot):
            pltpu.make_async_copy(dstT_v.at[slot, :, pl.ds(0, ROW_W)],
                                  out_hbm.at[pl.ds(0, IDX_W)],
                                  sem_o.at[slot]).wait()

        lane = lax.iota(jnp.int32, 16)
        vlanes = [lane + k * 16 for k in range(8)]

        def transpose_blk(slot, nk):
            src = src_v.at[slot]
            dstT = dstT_v.at[slot]

            @plsc.parallel_loop(0, EMB_DIM, step=1, unroll=4)
            def _(d):
                dvec = jnp.full((16,), d, jnp.int32)
                row = src.at[d]
                for k in range(nk):
                    plsc.store_scatter(dstT, [vlanes[k], dvec],
                                       row[pl.ds(k * 16, 16)])

        fire_in(blk_of(0), 0)
        fire_in(blk_of(1), 1)

        def outer(p, carry):
            for s in range(2):
                j = 2 * p + s
                wait_in(s)

                @pl.when(p > 0)
                def _():
                    wait_out(s)

                transpose_blk(s, 8)
                fire_out(blk_of(j), s)

                @pl.when(p < VBLK_PER_W // 2 - 1)
                def _():
                    fire_in(blk_of(j + 2), s)
            return carry

        lax.fori_loop(0, VBLK_PER_W // 2, outer, 0)
        wait_out(0)
        wait_out(1)

        @pl.when(wid < 4)
        def _():
            blk = VBLK_MAIN + wid
            pltpu.sync_copy(tabT_hbm.at[:, pl.ds(blk * IDX_W, IDX_W)],
                            src_v.at[0])
            transpose_blk(0, 8)
            pltpu.sync_copy(dstT_v.at[0, :, pl.ds(0, ROW_W)],
                            out_hbm.at[pl.ds(blk * IDX_W, IDX_W)])

        @pl.when(wid == 4)
        def _():
            v0 = N_VBLK * IDX_W
            pltpu.sync_copy(tail_hbm, src_v.at[0])
            transpose_blk(0, 4)
            pltpu.sync_copy(dstT_v.at[0, pl.ds(0, EMB_DIM), pl.ds(0, ROW_W)],
                            out_hbm.at[pl.ds(v0, EMB_DIM)])

    run = pl.kernel(
        body,
        out_type=jax.ShapeDtypeStruct((VOCAB_N, ROW_W), jnp.float32),
        mesh=mesh,
        scratch_types=[
            pltpu.VMEM((2, EMB_DIM, IDX_W), jnp.float32),
            pltpu.VMEM((2, IDX_W, ROW_W + 1), jnp.float32),
            pltpu.SemaphoreType.DMA((2,)),
            pltpu.SemaphoreType.DMA((2,)),
        ],
        compiler_params=pltpu.CompilerParams(use_tc_tiling_on_sc=True,
                                             needs_layout_passes=False),
    )
    return run(tableT, tail)


@functools.partial(jax.jit, static_argnames=("idx_per_worker",))
def _embed_lookup(x_idx, table_lin, idx_per_worker):
    mesh = plsc.VectorSubcoreMesh(core_axis_name="c", subcore_axis_name="s")
    info = plsc.get_sparse_core_info()
    nc = info.num_cores

    n_chunks = idx_per_worker // IDX_W
    n_outer = n_chunks // NSLOT

    def body(idx_hbm, table_hbm, out_hbm, idx_v, rows_v, rowsT_v, sem_g,
             sem_s):
        wid = lax.axis_index("s") * nc + lax.axis_index("c")
        base = wid * idx_per_worker
        pltpu.sync_copy(idx_hbm.at[pl.ds(base, idx_per_worker)], idx_v)

        def fire_gather(c, slot):
            pltpu.async_copy(table_hbm.at[idx_v.at[pl.ds(c * IDX_W, IDX_W)]],
                             rows_v.at[slot], sem_g.at[slot])

        def wait_gather(slot):
            pltpu.make_async_copy(table_hbm.at[idx_v.at[pl.ds(0, IDX_W)]],
                                  rows_v.at[slot], sem_g.at[slot]).wait()

        def fire_store(c, slot):
            g = base // IDX_W + c
            pltpu.async_copy(rowsT_v.at[slot, :, :, pl.ds(0, IDX_W)],
                             out_hbm.at[g // 32, :, g % 32],
                             sem_s.at[slot])

        def wait_store(slot):
            pltpu.make_async_copy(rowsT_v.at[slot, :, :, pl.ds(0, IDX_W)],
                                  out_hbm.at[0, :, 0],
                                  sem_s.at[slot]).wait()

        lane = lax.iota(jnp.int32, 16)
        d_hi = [(lane + k * 16) // 8 for k in range(4)]
        d_lo = lane % 8

        def transpose(slot):
            src = rows_v.at[slot]
            dstT = rowsT_v.at[slot]

            @plsc.parallel_loop(0, IDX_W, step=1, unroll=4)
            def _(b):
                bvec = jnp.full((16,), b, jnp.int32)
                row = src.at[b]
                for k in range(4):
                    plsc.store_scatter(dstT, [d_hi[k], d_lo, bvec],
                                       row[pl.ds(k * 16, 16)])

        for c in range(NSLOT):
            fire_gather(c, c)

        def outer(o, carry):
            for b in range(NSLOT):
                i = o * NSLOT + b
                wait_gather(b)

                @pl.when(o > 0)
                def _():
                    wait_store(b)

                transpose(b)
                fire_store(i, b)

                @pl.when(o < n_outer - 1)
                def _():
                    fire_gather(i + NSLOT, b)
            return carry

        lax.fori_loop(0, n_outer, outer, 0)

        for b in range(NSLOT):
            wait_store(b)

    run = pl.kernel(
        body,
        out_type=jax.ShapeDtypeStruct((200, 8, 32, 8, 128), jnp.float32),
        mesh=mesh,
        scratch_types=[
            pltpu.VMEM((idx_per_worker,), jnp.int32),
            pltpu.VMEM((NSLOT, IDX_W, ROW_W), jnp.float32),
            pltpu.VMEM((NSLOT, 8, 8, IDX_W + 1), jnp.float32),
            pltpu.SemaphoreType.DMA((NSLOT,)),
            pltpu.SemaphoreType.DMA((NSLOT,)),
        ],
        compiler_params=pltpu.CompilerParams(use_tc_tiling_on_sc=False,
                                             needs_layout_passes=False),
    )
    return run(x_idx, table_lin)


def kernel(x, table):
    b, l = x.shape
    total = b * l
    x_idx = x.T.reshape(total).astype(jnp.int32)
    tail = jnp.pad(table[N_VBLK * IDX_W:].T, ((0, 0), (0, EMB_DIM)))
    table_lin = _relayout(table.T, tail)
    info = plsc.get_sparse_core_info()
    n_workers = info.num_cores * info.num_subcores
    idx_per_worker = total // n_workers
    out5 = _embed_lookup(x_idx, table_lin, idx_per_worker)
    out = (out5.transpose(0, 1, 3, 2, 4)
           .reshape(l, EMB_DIM, b)
           .transpose(2, 0, 1))
    return out


# speedup vs baseline: 1.0730x; 1.0730x over previous
import functools

import jax
import jax.numpy as jnp
from jax import lax
from jax.experimental import pallas as pl
from jax.experimental.pallas import tpu as pltpu
from jax.experimental.pallas import tpu_sc as plsc

VOCAB_N = 1000000
EMB_DIM = 64
IDX_W = 128
NSLOT = 4

N_VBLK = VOCAB_N // IDX_W
N_WORKERS = 32
VBLK_MAIN = 7808
VBLK_PER_W = VBLK_MAIN // N_WORKERS


@jax.jit
def _relayout(tableT, tail):
    mesh = plsc.VectorSubcoreMesh(core_axis_name="c", subcore_axis_name="s")
    info = plsc.get_sparse_core_info()
    nc = info.num_cores

    def body(tabT_hbm, tail_hbm, out_hbm, src_v, dstT_v, sem_i, sem_o):
        wid = lax.axis_index("s") * nc + lax.axis_index("c")

        def blk_of(j):
            return j * N_WORKERS + wid

        def fire_in(blk, slot):
            pltpu.async_copy(tabT_hbm.at[:, pl.ds(blk * IDX_W, IDX_W)],
                             src_v.at[slot], sem_i.at[slot])

        def wait_in(slot):
            pltpu.make_async_copy(tabT_hbm.at[:, pl.ds(0, IDX_W)],
                                  src_v.at[slot], sem_i.at[slot]).wait()

        def fire_out(blk, slot):
            pltpu.async_copy(dstT_v.at[slot, :, pl.ds(0, 2 * EMB_DIM)],
                             out_hbm.at[pl.ds(blk * 64, 64)],
                             sem_o.at[slot])

        def wait_out(slot):
            pltpu.make_async_copy(dstT_v.at[slot, :, pl.ds(0, 2 * EMB_DIM)],
                                  out_hbm.at[pl.ds(0, 64)],
                                  sem_o.at[slot]).wait()

        lane = lax.iota(jnp.int32, 16)
        vlanes = [lane + k * 16 for k in range(8)]
        vhi = [v // 2 for v in vlanes]
        vlo = [(v % 2) * EMB_DIM for v in vlanes]

        def transpose_blk(slot, nk):
            src = src_v.at[slot]
            dstT = dstT_v.at[slot]

            @plsc.parallel_loop(0, EMB_DIM, step=1, unroll=4)
            def _(d):
                dvec = jnp.full((16,), d, jnp.int32)
                row = src.at[d]
                for k in range(nk):
                    plsc.store_scatter(dstT, [vhi[k], vlo[k] + dvec],
                                       row[pl.ds(k * 16, 16)])

        fire_in(blk_of(0), 0)
        fire_in(blk_of(1), 1)

        def outer(p, carry):
            for s in range(2):
                j = 2 * p + s
                wait_in(s)

                @pl.when(p > 0)
                def _():
                    wait_out(s)

                transpose_blk(s, 8)
                fire_out(blk_of(j), s)

                @pl.when(p < VBLK_PER_W // 2 - 1)
                def _():
                    fire_in(blk_of(j + 2), s)
            return carry

        lax.fori_loop(0, VBLK_PER_W // 2, outer, 0)
        wait_out(0)
        wait_out(1)

        @pl.when(wid < 4)
        def _():
            blk = VBLK_MAIN + wid
            pltpu.sync_copy(tabT_hbm.at[:, pl.ds(blk * IDX_W, IDX_W)],
                            src_v.at[0])
            transpose_blk(0, 8)
            pltpu.sync_copy(dstT_v.at[0, :, pl.ds(0, 2 * EMB_DIM)],
                            out_hbm.at[pl.ds(blk * 64, 64)])

        @pl.when(wid == 4)
        def _():
            r0 = N_VBLK * IDX_W // 2
            pltpu.sync_copy(tail_hbm, src_v.at[0])
            transpose_blk(0, 4)
            pltpu.sync_copy(dstT_v.at[0, pl.ds(0, 32), pl.ds(0, 2 * EMB_DIM)],
                            out_hbm.at[pl.ds(r0, 32)])

    run = pl.kernel(
        body,
        out_type=jax.ShapeDtypeStruct((VOCAB_N // 2, 2 * EMB_DIM),
                                      jnp.float32),
        mesh=mesh,
        scratch_types=[
            pltpu.VMEM((2, EMB_DIM, IDX_W), jnp.float32),
            pltpu.VMEM((2, 64, 2 * EMB_DIM + 1), jnp.float32),
            pltpu.SemaphoreType.DMA((2,)),
            pltpu.SemaphoreType.DMA((2,)),
        ],
        compiler_params=pltpu.CompilerParams(use_tc_tiling_on_sc=True,
                                             needs_layout_passes=False),
    )
    return run(tableT, tail)


@functools.partial(jax.jit, static_argnames=("idx_per_worker",))
def _embed_lookup(x_idx, table_lin, idx_per_worker):
    mesh = plsc.VectorSubcoreMesh(core_axis_name="c", subcore_axis_name="s")
    info = plsc.get_sparse_core_info()
    nc = info.num_cores

    n_chunks = idx_per_worker // IDX_W
    n_outer = n_chunks // NSLOT

    def body(idx_hbm, table_hbm, out_hbm, idx_v, rows_v, rowsT_v, sem_g,
             sem_s):
        wid = lax.axis_index("s") * nc + lax.axis_index("c")
        base = wid * idx_per_worker
        pltpu.sync_copy(idx_hbm.at[pl.ds(base, idx_per_worker)], idx_v)

        def fire_gather(c, slot):
            pltpu.async_copy(table_hbm.at[idx_v.at[pl.ds(c * IDX_W, IDX_W)]],
                             rows_v.at[slot], sem_g.at[slot])

        def wait_gather(slot):
            pltpu.make_async_copy(table_hbm.at[idx_v.at[pl.ds(0, IDX_W)]],
                                  rows_v.at[slot], sem_g.at[slot]).wait()

        def fire_store(c, slot):
            g = base // IDX_W + c
            pltpu.async_copy(rowsT_v.at[slot, :, :, pl.ds(0, IDX_W)],
                             out_hbm.at[g // 32, :, g % 32],
                             sem_s.at[slot])

        def wait_store(slot):
            pltpu.make_async_copy(rowsT_v.at[slot, :, :, pl.ds(0, IDX_W)],
                                  out_hbm.at[0, :, 0],
                                  sem_s.at[slot]).wait()

        lane = lax.iota(jnp.int32, 16)
        d_hi = [(lane + k * 16) // 8 for k in range(4)]
        d_lo = lane % 8

        def transpose(slot):
            src = rows_v.at[slot]
            dstT = rowsT_v.at[slot]

            @plsc.parallel_loop(0, IDX_W, step=1, unroll=4)
            def _(b):
                bvec = jnp.full((16,), b, jnp.int32)
                row = src.at[b]
                for k in range(4):
                    plsc.store_scatter(dstT, [d_hi[k], d_lo, bvec],
                                       row[pl.ds(k * 16, 16)])

        for c in range(NSLOT):
            fire_gather(c, c)

        def outer(o, carry):
            for b in range(NSLOT):
                i = o * NSLOT + b
                wait_gather(b)

                @pl.when(o > 0)
                def _():
                    wait_store(b)

                transpose(b)
                fire_store(i, b)

                @pl.when(o < n_outer - 1)
                def _():
                    fire_gather(i + NSLOT, b)
            return carry

        lax.fori_loop(0, n_outer, outer, 0)

        for b in range(NSLOT):
            wait_store(b)

    run = pl.kernel(
        body,
        out_type=jax.ShapeDtypeStruct((200, 8, 32, 8, 128), jnp.float32),
        mesh=mesh,
        scratch_types=[
            pltpu.VMEM((idx_per_worker,), jnp.int32),
            pltpu.VMEM((NSLOT, IDX_W, EMB_DIM), jnp.float32),
            pltpu.VMEM((NSLOT, 8, 8, IDX_W + 1), jnp.float32),
            pltpu.SemaphoreType.DMA((NSLOT,)),
            pltpu.SemaphoreType.DMA((NSLOT,)),
        ],
        compiler_params=pltpu.CompilerParams(use_tc_tiling_on_sc=False,
                                             needs_layout_passes=False),
    )
    return run(x_idx, table_lin.reshape(VOCAB_N, EMB_DIM))


def kernel(x, table):
    b, l = x.shape
    total = b * l
    x_idx = x.T.reshape(total).astype(jnp.int32)
    tail = jnp.pad(table[N_VBLK * IDX_W:].T, ((0, 0), (0, EMB_DIM)))
    table_lin = _relayout(table.T, tail)
    info = plsc.get_sparse_core_info()
    n_workers = info.num_cores * info.num_subcores
    idx_per_worker = total // n_workers
    out5 = _embed_lookup(x_idx, table_lin, idx_per_worker)
    out = (out5.transpose(0, 1, 3, 2, 4)
           .reshape(l, EMB_DIM, b)
           .transpose(2, 0, 1))
    return out


# speedup vs baseline: 1.0736x; 1.0006x over previous
import functools

import jax
import jax.numpy as jnp
from jax import lax
from jax.experimental import pallas as pl
from jax.experimental.pallas import tpu as pltpu
from jax.experimental.pallas import tpu_sc as plsc

VOCAB_N = 1000000
EMB_DIM = 64
IDX_W = 128
NSLOT = 5

N_VBLK = VOCAB_N // IDX_W
N_WORKERS = 32
VBLK_MAIN = 7808
VBLK_PER_W = VBLK_MAIN // N_WORKERS


@jax.jit
def _relayout(tableT, tail):
    mesh = plsc.VectorSubcoreMesh(core_axis_name="c", subcore_axis_name="s")
    info = plsc.get_sparse_core_info()
    nc = info.num_cores

    def body(tabT_hbm, tail_hbm, out_hbm, src_v, dstT_v, sem_i, sem_o):
        wid = lax.axis_index("s") * nc + lax.axis_index("c")

        def blk_of(j):
            return j * N_WORKERS + wid

        def fire_in(blk, slot):
            pltpu.async_copy(tabT_hbm.at[:, pl.ds(blk * IDX_W, IDX_W)],
                             src_v.at[slot], sem_i.at[slot])

        def wait_in(slot):
            pltpu.make_async_copy(tabT_hbm.at[:, pl.ds(0, IDX_W)],
                                  src_v.at[slot], sem_i.at[slot]).wait()

        def fire_out(blk, slot):
            pltpu.async_copy(dstT_v.at[slot, :, pl.ds(0, 2 * EMB_DIM)],
                             out_hbm.at[pl.ds(blk * 64, 64)],
                             sem_o.at[slot])

        def wait_out(slot):
            pltpu.make_async_copy(dstT_v.at[slot, :, pl.ds(0, 2 * EMB_DIM)],
                                  out_hbm.at[pl.ds(0, 64)],
                                  sem_o.at[slot]).wait()

        lane = lax.iota(jnp.int32, 16)
        vlanes = [lane + k * 16 for k in range(8)]
        vhi = [v // 2 for v in vlanes]
        vlo = [(v % 2) * EMB_DIM for v in vlanes]

        def transpose_blk(slot, nk):
            src = src_v.at[slot]
            dstT = dstT_v.at[slot]

            @plsc.parallel_loop(0, EMB_DIM, step=1, unroll=4)
            def _(d):
                dvec = jnp.full((16,), d, jnp.int32)
                row = src.at[d]
                for k in range(nk):
                    plsc.store_scatter(dstT, [vhi[k], vlo[k] + dvec],
                                       row[pl.ds(k * 16, 16)])

        fire_in(blk_of(0), 0)
        fire_in(blk_of(1), 1)

        def outer(p, carry):
            for s in range(2):
                j = 2 * p + s
                wait_in(s)

                @pl.when(p > 0)
                def _():
                    wait_out(s)

                transpose_blk(s, 8)
                fire_out(blk_of(j), s)

                @pl.when(p < VBLK_PER_W // 2 - 1)
                def _():
                    fire_in(blk_of(j + 2), s)
            return carry

        lax.fori_loop(0, VBLK_PER_W // 2, outer, 0)
        wait_out(0)
        wait_out(1)

        @pl.when(wid < 4)
        def _():
            blk = VBLK_MAIN + wid
            pltpu.sync_copy(tabT_hbm.at[:, pl.ds(blk * IDX_W, IDX_W)],
                            src_v.at[0])
            transpose_blk(0, 8)
            pltpu.sync_copy(dstT_v.at[0, :, pl.ds(0, 2 * EMB_DIM)],
                            out_hbm.at[pl.ds(blk * 64, 64)])

        @pl.when(wid == 4)
        def _():
            r0 = N_VBLK * IDX_W // 2
            pltpu.sync_copy(tail_hbm, src_v.at[0])
            transpose_blk(0, 4)
            pltpu.sync_copy(dstT_v.at[0, pl.ds(0, 32), pl.ds(0, 2 * EMB_DIM)],
                            out_hbm.at[pl.ds(r0, 32)])

    run = pl.kernel(
        body,
        out_type=jax.ShapeDtypeStruct((VOCAB_N // 2, 2 * EMB_DIM),
                                      jnp.float32),
        mesh=mesh,
        scratch_types=[
            pltpu.VMEM((2, EMB_DIM, IDX_W), jnp.float32),
            pltpu.VMEM((2, 64, 2 * EMB_DIM + 1), jnp.float32),
            pltpu.SemaphoreType.DMA((2,)),
            pltpu.SemaphoreType.DMA((2,)),
        ],
        compiler_params=pltpu.CompilerParams(use_tc_tiling_on_sc=True,
                                             needs_layout_passes=False),
    )
    return run(tableT, tail)


@functools.partial(jax.jit, static_argnames=("idx_per_worker",))
def _embed_lookup(x_idx, table_lin, idx_per_worker):
    mesh = plsc.VectorSubcoreMesh(core_axis_name="c", subcore_axis_name="s")
    info = plsc.get_sparse_core_info()
    nc = info.num_cores

    n_chunks = idx_per_worker // IDX_W
    n_outer = n_chunks // NSLOT

    def body(idx_hbm, table_hbm, out_hbm, idx_v, rows_v, rowsT_v, sem_g,
             sem_s):
        wid = lax.axis_index("s") * nc + lax.axis_index("c")
        base = wid * idx_per_worker
        pltpu.sync_copy(idx_hbm.at[pl.ds(base, idx_per_worker)], idx_v)

        def fire_gather(c, slot):
            pltpu.async_copy(table_hbm.at[idx_v.at[pl.ds(c * IDX_W, IDX_W)]],
                             rows_v.at[slot], sem_g.at[slot])

        def wait_gather(slot):
            pltpu.make_async_copy(table_hbm.at[idx_v.at[pl.ds(0, IDX_W)]],
                                  rows_v.at[slot], sem_g.at[slot]).wait()

        def fire_store(c, slot):
            g = base // IDX_W + c
            pltpu.async_copy(rowsT_v.at[slot, :, :, pl.ds(0, IDX_W)],
                             out_hbm.at[g // 32, :, g % 32],
                             sem_s.at[slot])

        def wait_store(slot):
            pltpu.make_async_copy(rowsT_v.at[slot, :, :, pl.ds(0, IDX_W)],
                                  out_hbm.at[0, :, 0],
                                  sem_s.at[slot]).wait()

        lane = lax.iota(jnp.int32, 16)
        d_hi = [(lane + k * 16) // 8 for k in range(4)]
        d_lo = lane % 8

        def transpose(slot):
            src = rows_v.at[slot]
            dstT = rowsT_v.at[slot]

            @plsc.parallel_loop(0, IDX_W, step=1, unroll=4)
            def _(b):
                bvec = jnp.full((16,), b, jnp.int32)
                row = src.at[b]
                for k in range(4):
                    plsc.store_scatter(dstT, [d_hi[k], d_lo, bvec],
                                       row[pl.ds(k * 16, 16)])

        for c in range(NSLOT):
            fire_gather(c, c)

        def outer(o, carry):
            for b in range(NSLOT):
                i = o * NSLOT + b
                wait_gather(b)

                @pl.when(o > 0)
                def _():
                    wait_store(b)

                transpose(b)
                fire_store(i, b)

                @pl.when(o < n_outer - 1)
                def _():
                    fire_gather(i + NSLOT, b)
            return carry

        lax.fori_loop(0, n_outer, outer, 0)

        for b in range(NSLOT):
            wait_store(b)

    run = pl.kernel(
        body,
        out_type=jax.ShapeDtypeStruct((200, 8, 32, 8, 128), jnp.float32),
        mesh=mesh,
        scratch_types=[
            pltpu.VMEM((idx_per_worker,), jnp.int32),
            pltpu.VMEM((NSLOT, IDX_W, EMB_DIM), jnp.float32),
            pltpu.VMEM((NSLOT, 8, 8, IDX_W + 1), jnp.float32),
            pltpu.SemaphoreType.DMA((NSLOT,)),
            pltpu.SemaphoreType.DMA((NSLOT,)),
        ],
        compiler_params=pltpu.CompilerParams(use_tc_tiling_on_sc=False,
                                             needs_layout_passes=False),
    )
    return run(x_idx, table_lin.reshape(VOCAB_N, EMB_DIM))


def kernel(x, table):
    b, l = x.shape
    total = b * l
    x_idx = x.T.reshape(total).astype(jnp.int32)
    tail = jnp.pad(table[N_VBLK * IDX_W:].T, ((0, 0), (0, EMB_DIM)))
    table_lin = _relayout(table.T, tail)
    info = plsc.get_sparse_core_info()
    n_workers = info.num_cores * info.num_subcores
    idx_per_worker = total // n_workers
    out5 = _embed_lookup(x_idx, table_lin, idx_per_worker)
    out = (out5.transpose(0, 1, 3, 2, 4)
           .reshape(l, EMB_DIM, b)
           .transpose(2, 0, 1))
    return out
